# baseline (speedup 1.0000x reference)
import functools

import jax
import jax.numpy as jnp
from jax import lax
from jax.experimental import pallas as pl
from jax.experimental.pallas import tpu as pltpu
from jax.experimental.pallas import tpu_sc as plsc

BS, N, IN_DIM, HID = 1024, 128, 128, 128
BB = 32


def _tc_dense_kernel(nf_ref, adj_ref, w_ref, mm_ref, ideg_ref, odeg_ref):
    nf = nf_ref[...]
    adj = adj_ref[...]
    w = w_ref[...]
    mm = jnp.dot(nf.reshape(BB * N, IN_DIM), w,
                 preferred_element_type=jnp.float32)
    mm_ref[...] = mm
    ideg_ref[...] = jnp.floor(jnp.sum(adj, axis=2)).astype(jnp.int32)
    odeg_ref[...] = jnp.floor(jnp.sum(adj, axis=1)).astype(jnp.int32)


def _tc_dense(node_features, adj, W_atom):
    grid = (BS // BB,)
    return pl.pallas_call(
        _tc_dense_kernel,
        grid=grid,
        in_specs=[
            pl.BlockSpec((BB, N, IN_DIM), lambda i: (i, 0, 0)),
            pl.BlockSpec((BB, N, N), lambda i: (i, 0, 0)),
            pl.BlockSpec((IN_DIM, HID), lambda i: (0, 0)),
        ],
        out_specs=[
            pl.BlockSpec((BB * N, HID), lambda i: (i, 0)),
            pl.BlockSpec((BB, N), lambda i: (i, 0)),
            pl.BlockSpec((BB, N), lambda i: (i, 0)),
        ],
        out_shape=[
            jax.ShapeDtypeStruct((BS * N, HID), jnp.float32),
            jax.ShapeDtypeStruct((BS, N), jnp.int32),
            jax.ShapeDtypeStruct((BS, N), jnp.int32),
        ],
        compiler_params=pltpu.CompilerParams(
            dimension_semantics=("parallel",),
        ),
    )(node_features, adj, W_atom)


TAB_ROWS = 136
NSLOT = 3


def _sc_lookup_body(mm_hbm, ideg_hbm, odeg_hbm, in_tab_hbm, out_tab_hbm,
                    tok_hbm, out_hbm,
                    ideg_v, odeg_v, in_tab_v, out_tab_v, tok_v,
                    stage0, stage1, stage2,
                    sem_ld0, sem_ld1, sem_ld2, sem_st0, sem_st1, sem_st2):
    info = plsc.get_sparse_core_info()
    nw = info.num_cores * info.num_subcores
    wid = lax.axis_index("s") * info.num_cores + lax.axis_index("c")
    batches_per_w = BS // nw
    b0 = wid * batches_per_w
    r0 = b0 * N

    pre = [
        pltpu.async_copy(ideg_hbm.at[pl.ds(r0, batches_per_w * N)], ideg_v,
                         sem_st0),
        pltpu.async_copy(odeg_hbm.at[pl.ds(r0, batches_per_w * N)], odeg_v,
                         sem_st0),
        pltpu.async_copy(in_tab_hbm.at[pl.ds(0, TAB_ROWS)], in_tab_v,
                         sem_st1),
        pltpu.async_copy(out_tab_hbm.at[pl.ds(0, TAB_ROWS)], out_tab_v,
                         sem_st1),
    ]
    pltpu.sync_copy(tok_hbm, tok_v)

    stages = [stage0, stage1, stage2]
    sem_ld = [sem_ld0, sem_ld1, sem_ld2]
    sem_st = [sem_st0, sem_st1, sem_st2]

    for st in stages:
        for c in range(HID // 16):
            sl = pl.ds(c * 16, 16)
            st[0, sl] = tok_v[0, sl]

    col_iota = lax.iota(jnp.int32, 16)
    col_vecs = [col_iota + (c * 16) for c in range(HID // 16)]

    def issue_load(i, slot):
        return pltpu.async_copy(
            mm_hbm.at[pl.ds((b0 + i) * N, N)],
            stages[slot].at[pl.ds(1, N)],
            sem_ld[slot])

    def wait_load(slot):
        pltpu.make_async_copy(
            mm_hbm.at[pl.ds(0, N)], stages[slot].at[pl.ds(1, N)],
            sem_ld[slot]).wait()

    def wait_store(slot):
        pltpu.make_async_copy(
            stages[slot], out_hbm.at[0], sem_st[slot]).wait()

    def compute_batch(i, slot):
        st = stages[slot]
        base = i * N

        @plsc.parallel_loop(0, N, 1, unroll=8)
        def row_body(r):
            deg_i = plsc.load_gather(ideg_v, [jnp.full((16,), base + r,
                                                       jnp.int32)])
            deg_o = plsc.load_gather(odeg_v, [jnp.full((16,), base + r,
                                                       jnp.int32)])
            for c in range(HID // 16):
                g = (plsc.load_gather(in_tab_v, [deg_i, col_vecs[c]]) +
                     plsc.load_gather(out_tab_v, [deg_o, col_vecs[c]]))
                plsc.addupdate(st.at[1 + r, pl.ds(c * 16, 16)], g)

    issue_load(0, 0)
    for cp in pre:
        cp.wait()

    def loop_body(i, carry):
        for k in range(NSLOT):
            nxt = (k + 1) % NSLOT

            @pl.when(lax.rem(i, NSLOT) == k)
            def _():
                @pl.when(i >= 2)
                def _():
                    wait_store(nxt)

                @pl.when(i + 1 < batches_per_w)
                def _():
                    issue_load(i + 1, nxt)

                wait_load(k)
                compute_batch(i, k)
                pltpu.async_copy(stages[k], out_hbm.at[b0 + i], sem_st[k])

        return carry

    lax.fori_loop(0, batches_per_w, loop_body, 0)
    wait_store((batches_per_w - 2) % NSLOT)
    wait_store((batches_per_w - 1) % NSLOT)


def _sc_lookup(mm, ideg, odeg, in_table, out_table, graph_token):
    mesh = plsc.VectorSubcoreMesh(core_axis_name="c", subcore_axis_name="s")
    bpw = BS // 32
    kern = pl.kernel(
        _sc_lookup_body,
        out_type=jax.ShapeDtypeStruct((BS, N + 1, HID), jnp.float32),
        mesh=mesh,
        compiler_params=pltpu.CompilerParams(needs_layout_passes=False),
        scratch_types=[
            pltpu.VMEM((bpw * N,), jnp.int32),
            pltpu.VMEM((bpw * N,), jnp.int32),
            pltpu.VMEM((TAB_ROWS, HID), jnp.float32),
            pltpu.VMEM((TAB_ROWS, HID), jnp.float32),
            pltpu.VMEM((1, HID), jnp.float32),
            pltpu.VMEM((N + 1, HID), jnp.float32),
            pltpu.VMEM((N + 1, HID), jnp.float32),
            pltpu.VMEM((N + 1, HID), jnp.float32),
            pltpu.SemaphoreType.DMA,
            pltpu.SemaphoreType.DMA,
            pltpu.SemaphoreType.DMA,
            pltpu.SemaphoreType.DMA,
            pltpu.SemaphoreType.DMA,
            pltpu.SemaphoreType.DMA,
        ],
    )
    return kern(mm, ideg, odeg, in_table, out_table, graph_token)


@jax.jit
def kernel(node_features, adj, W_atom, in_table, out_table, graph_token):
    mm, ideg, odeg = _tc_dense(node_features, adj, W_atom)
    return _sc_lookup(mm, ideg.reshape(BS * N), odeg.reshape(BS * N),
                      in_table, out_table, graph_token)


# speedup vs baseline: 1.0110x; 1.0110x over previous
import jax
import jax.numpy as jnp
from jax import lax
from jax.experimental import pallas as pl
from jax.experimental.pallas import tpu as pltpu
from jax.experimental.pallas import tpu_sc as plsc

BS, N, IN_DIM, HID = 1024, 128, 128, 128
BB = 32


def _tc_dense_kernel(nf_ref, adj_ref, w_ref, mm_ref, ideg_ref, odeg_ref):
    nf = nf_ref[...]
    adj = adj_ref[...]
    w = w_ref[...]
    mm = jnp.dot(nf.reshape(BB * N, IN_DIM), w,
                 preferred_element_type=jnp.float32)
    mm_ref[...] = mm
    ideg_ref[...] = jnp.floor(jnp.sum(adj, axis=2)).astype(jnp.int32)
    odeg_ref[...] = jnp.floor(jnp.sum(adj, axis=1)).astype(jnp.int32)


def _tc_dense(node_features, adj, W_atom):
    grid = (BS // BB,)
    return pl.pallas_call(
        _tc_dense_kernel,
        grid=grid,
        in_specs=[
            pl.BlockSpec((BB, N, IN_DIM), lambda i: (i, 0, 0)),
            pl.BlockSpec((BB, N, N), lambda i: (i, 0, 0)),
            pl.BlockSpec((IN_DIM, HID), lambda i: (0, 0)),
        ],
        out_specs=[
            pl.BlockSpec((BB * N, HID), lambda i: (i, 0)),
            pl.BlockSpec((BB, N), lambda i: (i, 0)),
            pl.BlockSpec((BB, N), lambda i: (i, 0)),
        ],
        out_shape=[
            jax.ShapeDtypeStruct((BS * N, HID), jnp.float32),
            jax.ShapeDtypeStruct((BS, N), jnp.int32),
            jax.ShapeDtypeStruct((BS, N), jnp.int32),
        ],
        compiler_params=pltpu.CompilerParams(
            dimension_semantics=("parallel",),
        ),
    )(node_features, adj, W_atom)


TAB_ROWS = 136
CHUNK_B = 2
STAGE_ROWS = CHUNK_B * (N + 1)


def _sc_lookup_body(mm_hbm, ideg_hbm, odeg_hbm, in_tab_hbm, out_tab_hbm,
                    tok_hbm, out_hbm,
                    ideg_v, odeg_v, in_tab_v, out_tab_v, tok_v,
                    stage0, stage1, sem_ld0, sem_ld1, sem_st0, sem_st1):
    info = plsc.get_sparse_core_info()
    nw = info.num_cores * info.num_subcores
    wid = lax.axis_index("s") * info.num_cores + lax.axis_index("c")
    batches_per_w = BS // nw
    n_chunks = batches_per_w // CHUNK_B
    b0 = wid * batches_per_w
    r0 = b0 * N

    pre = [
        pltpu.async_copy(ideg_hbm.at[pl.ds(r0, batches_per_w * N)], ideg_v,
                         sem_st0),
        pltpu.async_copy(odeg_hbm.at[pl.ds(r0, batches_per_w * N)], odeg_v,
                         sem_st0),
        pltpu.async_copy(in_tab_hbm.at[pl.ds(0, TAB_ROWS)], in_tab_v,
                         sem_st1),
        pltpu.async_copy(out_tab_hbm.at[pl.ds(0, TAB_ROWS)], out_tab_v,
                         sem_st1),
    ]
    pltpu.sync_copy(tok_hbm, tok_v)

    stages = [stage0, stage1]
    sem_ld = [sem_ld0, sem_ld1]
    sem_st = [sem_st0, sem_st1]

    for st in stages:
        for p in range(CHUNK_B):
            for c in range(HID // 16):
                sl = pl.ds(c * 16, 16)
                st[p * (N + 1), sl] = tok_v[0, sl]

    col_iota = lax.iota(jnp.int32, 16)

    def issue_loads(ci, slot):
        cps = []
        for p in range(CHUNK_B):
            b = b0 + ci * CHUNK_B + p
            cps.append(pltpu.async_copy(
                mm_hbm.at[pl.ds(b * N, N)],
                stages[slot].at[pl.ds(p * (N + 1) + 1, N)],
                sem_ld[slot]))
        return cps

    def issue_stores(ci, slot):
        cps = []
        for p in range(CHUNK_B):
            b = b0 + ci * CHUNK_B + p
            cps.append(pltpu.async_copy(
                stages[slot].at[pl.ds(p * (N + 1), N + 1)],
                out_hbm.at[b],
                sem_st[slot]))
        return cps

    col_vecs = [col_iota + (c * 16) for c in range(HID // 16)]

    def compute_chunk(ci, slot):
        st = stages[slot]
        base = ci * CHUNK_B * N

        @plsc.parallel_loop(0, CHUNK_B * N, 1, unroll=8)
        def row_body(r):
            deg_i = plsc.load_gather(ideg_v, [jnp.full((16,), base + r,
                                                       jnp.int32)])
            deg_o = plsc.load_gather(odeg_v, [jnp.full((16,), base + r,
                                                       jnp.int32)])
            srow = r + 1 + lax.shift_right_logical(r, 7)
            for c in range(HID // 16):
                g = (plsc.load_gather(in_tab_v, [deg_i, col_vecs[c]]) +
                     plsc.load_gather(out_tab_v, [deg_o, col_vecs[c]]))
                plsc.addupdate(st.at[srow, pl.ds(c * 16, 16)], g)

    ld_cps = {0: issue_loads(0, 0)}
    for cp in pre:
        cp.wait()
    st_cps = {}
    for ci in range(n_chunks):
        slot = ci % 2
        nxt = (ci + 1) % 2
        if ci + 1 < n_chunks:
            if ci >= 1:
                for cp in st_cps[ci - 1]:
                    cp.wait()
            ld_cps[ci + 1] = issue_loads(ci + 1, nxt)
        for cp in ld_cps[ci]:
            cp.wait()
        compute_chunk(ci, slot)
        st_cps[ci] = issue_stores(ci, slot)
    for cp in st_cps[n_chunks - 2]:
        cp.wait()
    for cp in st_cps[n_chunks - 1]:
        cp.wait()


def _sc_lookup(mm, ideg, odeg, in_table, out_table, graph_token):
    mesh = plsc.VectorSubcoreMesh(core_axis_name="c", subcore_axis_name="s")
    bpw = BS // 32
    kern = pl.kernel(
        _sc_lookup_body,
        out_type=jax.ShapeDtypeStruct((BS, N + 1, HID), jnp.float32),
        mesh=mesh,
        compiler_params=pltpu.CompilerParams(needs_layout_passes=False),
        scratch_types=[
            pltpu.VMEM((bpw * N,), jnp.int32),
            pltpu.VMEM((bpw * N,), jnp.int32),
            pltpu.VMEM((TAB_ROWS, HID), jnp.float32),
            pltpu.VMEM((TAB_ROWS, HID), jnp.float32),
            pltpu.VMEM((1, HID), jnp.float32),
            pltpu.VMEM((STAGE_ROWS, HID), jnp.float32),
            pltpu.VMEM((STAGE_ROWS, HID), jnp.float32),
            pltpu.SemaphoreType.DMA,
            pltpu.SemaphoreType.DMA,
            pltpu.SemaphoreType.DMA,
            pltpu.SemaphoreType.DMA,
        ],
    )
    return kern(mm, ideg, odeg, in_table, out_table, graph_token)


@jax.jit
def kernel(node_features, adj, W_atom, in_table, out_table, graph_token):
    mm, ideg, odeg = _tc_dense(node_features, adj, W_atom)
    return _sc_lookup(mm, ideg.reshape(BS * N), odeg.reshape(BS * N),
                      in_table, out_table, graph_token)
